# baseline (speedup 1.0000x reference)
import dataclasses
import functools

import jax
import jax.numpy as jnp
import numpy as np
from jax import lax
from jax.experimental import pallas as pl
from jax.experimental.pallas import tpu as pltpu
from jax.experimental.pallas import tpu_sc as plsc

N = 10000
E = 160000
D = 128
EF = 40
MAXN = 12
NG = 100
NATOM = 100
AENC = 200
DELTA = float(np.log(MAXN + 1.0))
NEG = np.float32(-3.0e38)

EB = 6400
W = 800
NWIN = E // W
VPW = W // 16
CHUNK = E // 32


def _node_prep_body(atom_ref, dist_ref, af_ref, Wa_ref, ba_ref, Wd_ref,
                    bd_ref, g_ref, b_ref, Wp1_ref, Wp2_ref,
                    h_ref, aT_ref, bT_ref):
    atom = atom_ref[...]
    onehot = (lax.broadcasted_iota(jnp.int32, (N, NATOM), 1)
              == atom[:, None]).astype(jnp.float32)
    table2 = jnp.dot(af_ref[...], Wa_ref[...],
                     preferred_element_type=jnp.float32)
    hp = jnp.dot(onehot, table2, preferred_element_type=jnp.float32)
    hp = hp + ba_ref[...][None, :]
    hp = hp + jnp.dot(dist_ref[...], Wd_ref[...],
                      preferred_element_type=jnp.float32) + bd_ref[...][None, :]
    mu = jnp.mean(hp, axis=1, keepdims=True)
    va = jnp.mean((hp - mu) * (hp - mu), axis=1, keepdims=True)
    h = (hp - mu) * lax.rsqrt(va + 1e-5) * g_ref[...][None, :] + b_ref[...][None, :]
    h_ref[...] = h
    aT_ref[...] = lax.dot_general(Wp1_ref[...], h, (((0,), (1,)), ((), ())),
                                  preferred_element_type=jnp.float32)
    bT_ref[...] = lax.dot_general(Wp2_ref[...], h, (((0,), (1,)), ((), ())),
                                  preferred_element_type=jnp.float32)


def _node_prep(atom, dist, af_table, W_atom, b_atom, W_dist, b_dist,
               ln_g, ln_b, Wp1, Wp2):
    return pl.pallas_call(
        _node_prep_body,
        out_shape=[jax.ShapeDtypeStruct((N, D), jnp.float32),
                   jax.ShapeDtypeStruct((D, N), jnp.float32),
                   jax.ShapeDtypeStruct((D, N), jnp.float32)],
    )(atom, dist, af_table, W_atom, b_atom, W_dist, b_dist, ln_g, ln_b,
      Wp1, Wp2)


_CENTERS = np.linspace(0.0, 1.0, EF).astype(np.float32)
_GAMMA = np.float32(1.0 / (_CENTERS[1] - _CENTERS[0]) ** 2)


def _edge_prep_body(r_ref, We_ref, be_ref, Wp30_ref, bp0_ref, Wp31_ref,
                    bp1_ref, c0_ref, c1_ref):
    r = r_ref[...]
    d2 = jnp.sum(r * r, axis=0)
    invd = 1.0 / jnp.sqrt(d2)
    centers = (lax.broadcasted_iota(jnp.int32, (EF, 1), 0)
               .astype(jnp.float32) * np.float32(1.0 / (EF - 1)))
    gT = jnp.exp(-_GAMMA * (invd[None, :] - centers) ** 2)
    for (Wp3_ref, bp_ref, c_ref) in ((Wp30_ref, bp0_ref, c0_ref),
                                     (Wp31_ref, bp1_ref, c1_ref)):
        U = jnp.dot(We_ref[...], Wp3_ref[...],
                    preferred_element_type=jnp.float32)
        cst = jnp.dot(be_ref[...][None, :], Wp3_ref[...],
                      preferred_element_type=jnp.float32)[0] + bp_ref[...]
        c_ref[...] = lax.dot_general(U, gT, (((0,), (0,)), ((), ())),
                                     preferred_element_type=jnp.float32) \
            + cst[:, None]


def _edge_prep(r, W_edge, b_edge, Wp30, bp0, Wp31, bp1):
    nblk = E // EB
    return pl.pallas_call(
        _edge_prep_body,
        grid=(nblk,),
        in_specs=[
            pl.BlockSpec((3, EB), lambda i: (0, i)),
            pl.BlockSpec((EF, D), lambda i: (0, 0)),
            pl.BlockSpec((D,), lambda i: (0,)),
            pl.BlockSpec((D, D), lambda i: (0, 0)),
            pl.BlockSpec((D,), lambda i: (0,)),
            pl.BlockSpec((D, D), lambda i: (0, 0)),
            pl.BlockSpec((D,), lambda i: (0,)),
        ],
        out_specs=[pl.BlockSpec((D, EB), lambda i: (0, i)),
                   pl.BlockSpec((D, EB), lambda i: (0, i))],
        out_shape=[jax.ShapeDtypeStruct((D, E), jnp.float32),
                   jax.ShapeDtypeStruct((D, E), jnp.float32)],
    )(r.T, W_edge, b_edge, Wp30, bp0, Wp31, bp1)


def _post_body(h_ref, s_ref, q_ref, x_ref, n_ref, degp_ref, Wq_ref, bq_ref,
               out_ref):
    deg = degp_ref[0, :] + degp_ref[1, :]
    degc = jnp.maximum(deg, 1.0)
    inv = 1.0 / degc
    meanT = s_ref[...] * inv[None, :]
    msqT = q_ref[...] * inv[None, :]
    varT = jnp.maximum(msqT - meanT * meanT, 0.0)
    pos = (deg > 0.0)[None, :]
    mxT = jnp.where(pos, x_ref[...], 0.0)
    mnT = jnp.where(pos, -n_ref[...], 0.0)
    att = (DELTA / jnp.log(degc + 1.0))[:, None]
    Wq = Wq_ref[...]

    def dgT(xT, k):
        return lax.dot_general(xT, Wq[D * k:D * (k + 1)],
                               (((0,), (0,)), ((), ())),
                               preferred_element_type=jnp.float32)

    out = jnp.dot(h_ref[...], Wq[:D], preferred_element_type=jnp.float32)
    out = out + dgT(meanT, 1) + att * dgT(meanT, 5)
    out = out + dgT(varT, 2) + att * dgT(varT, 6)
    out = out + dgT(mnT, 3) + att * dgT(mnT, 7)
    out = out + dgT(mxT, 4) + att * dgT(mxT, 8)
    out_ref[...] = out + bq_ref[...][None, :]


def _post(h, s, q, x, n, degp, Wq, bq):
    return pl.pallas_call(
        _post_body,
        out_shape=jax.ShapeDtypeStruct((N, D), jnp.float32),
    )(h, s, q, x, n, degp, Wq, bq)


def _abt_body(h_ref, Wp1_ref, Wp2_ref, aT_ref, bT_ref):
    h = h_ref[...]
    aT_ref[...] = lax.dot_general(Wp1_ref[...], h, (((0,), (1,)), ((), ())),
                                  preferred_element_type=jnp.float32)
    bT_ref[...] = lax.dot_general(Wp2_ref[...], h, (((0,), (1,)), ((), ())),
                                  preferred_element_type=jnp.float32)


def _abt(h, Wp1, Wp2):
    return pl.pallas_call(
        _abt_body,
        out_shape=[jax.ShapeDtypeStruct((D, N), jnp.float32),
                   jax.ShapeDtypeStruct((D, N), jnp.float32)],
    )(h, Wp1, Wp2)


def _readout_body(h_ref, gid_ref, w_ref, b_ref, o_ref):
    h = h_ref[...]
    gid = gid_ref[...]
    seg = lax.broadcasted_iota(jnp.int32, (NG, N), 0)
    onehot = (seg == gid[None, :]).astype(jnp.float32)
    s = jnp.dot(onehot, h, preferred_element_type=jnp.float32)
    cnt = jnp.maximum(jnp.sum(onehot, axis=1, keepdims=True), 1.0)
    pooled = s / cnt
    o_ref[...] = jnp.dot(pooled, w_ref[...],
                         preferred_element_type=jnp.float32) + b_ref[...][None, :]


def _readout(h, gid, W_out, b_out):
    return pl.pallas_call(
        _readout_body,
        out_shape=jax.ShapeDtypeStruct((NG, 1), jnp.float32),
    )(h, gid, W_out, b_out)


def _deg_sc(dst):
    mesh = plsc.VectorSubcoreMesh(core_axis_name="c", subcore_axis_name="s")

    @functools.partial(
        pl.kernel, mesh=mesh,
        out_type=jax.ShapeDtypeStruct((2 * N,), jnp.float32),
        scratch_types=[
            pltpu.VMEM((CHUNK,), jnp.int32),
            pltpu.VMEM((CHUNK,), jnp.float32),
            pltpu.VMEM_SHARED((N,), jnp.float32),
            pltpu.SemaphoreType.DMA,
        ])
    def k(dst_hbm, deg_hbm, dstv, onesv, shared_deg, sem):
        c = lax.axis_index("c")
        s = lax.axis_index("s")

        @pl.loop(0, CHUNK, step=16)
        def _fill(i):
            onesv[pl.ds(i, 16)] = jnp.zeros((16,), jnp.float32)

        @pl.when(s == 0)
        def _zero():
            pltpu.sync_copy(onesv, shared_deg.at[pl.ds(0, CHUNK)])
            pltpu.sync_copy(onesv, shared_deg.at[pl.ds(CHUNK, CHUNK)])

        @pl.loop(0, CHUNK, step=16)
        def _fill1(i):
            onesv[pl.ds(i, 16)] = jnp.ones((16,), jnp.float32)

        plsc.subcore_barrier()
        base = (c * 16 + s) * CHUNK
        pltpu.async_copy(dst_hbm.at[pl.ds(base, CHUNK)], dstv, sem).wait()
        pltpu.sync_copy(onesv, shared_deg.at[dstv], add=True)
        plsc.subcore_barrier()

        @pl.when(s == 0)
        def _out():
            for half in (0, 1):
                pltpu.sync_copy(shared_deg.at[pl.ds(half * CHUNK, CHUNK)],
                                onesv)
                pltpu.sync_copy(onesv,
                                deg_hbm.at[pl.ds(c * N + half * CHUNK, CHUNK)])

    return k(dst).reshape(2, N)


_IOTA16 = np.arange(16, dtype=np.int32)


def _edge_stage(aT, bT, cT, src, dst):
    mesh = plsc.VectorSubcoreMesh(core_axis_name="c", subcore_axis_name="s")

    vm = pltpu.VMEM
    scratch = ([vm((N,), jnp.float32)] * 8
               + [vm((N,), jnp.float32)] * 4
               + [vm((W,), jnp.int32)] * 4
               + [vm((W,), jnp.float32)] * 4
               + [vm((16,), jnp.int32), vm((16,), jnp.float32)]
               + [pltpu.SemaphoreType.DMA, pltpu.SemaphoreType.DMA])

    cp = pltpu.CompilerParams()
    if "needs_layout_passes" in pltpu.CompilerParams.__dataclass_fields__:
        cp = dataclasses.replace(cp, needs_layout_passes=False)

    @functools.partial(
        pl.kernel, mesh=mesh,
        out_type=[jax.ShapeDtypeStruct((D * N,), jnp.float32)] * 4,
        scratch_types=scratch, compiler_params=cp)
    def k(aT_hbm, bT_hbm, cT_hbm, src_hbm, dst_hbm,
          osum, osq, omx, omn,
          as0, as1, aq0, aq1, ax0, ax1, an0, an1,
          ac0, ac1, bc0, bc1,
          sb0, sb1, db0, db1,
          cb00, cb01, cb10, cb11,
          sk16, sv16, sem0, sem1):
        cidx = lax.axis_index("c")
        sidx = lax.axis_index("s")
        wid = sidx * 2 + cidx

        iota = lax.iota(jnp.int32, 16)
        sh1 = jnp.maximum(iota - 1, 0)
        sh2 = jnp.maximum(iota - 2, 0)
        sh4 = jnp.maximum(iota - 4, 0)
        sh8 = jnp.maximum(iota - 8, 0)
        shp1 = jnp.minimum(iota + 1, 15)
        ge1 = iota >= 1
        ge2 = iota >= 2
        ge4 = iota >= 4
        ge8 = iota >= 8
        is15 = iota == 15
        shifts = ((sh1, ge1), (sh2, ge2), (sh4, ge4), (sh8, ge8))

        accs = ((as0, aq0, ax0, an0), (as1, aq1, ax1, an1))
        acols = (ac0, ac1)
        bcols = (bc0, bc1)
        sbufs = (sb0, sb1)
        dbufs = (db0, db1)
        cbufs = ((cb00, cb01), (cb10, cb11))
        sems = (sem0, sem1)

        def issue(slot, w, c0):
            off = w * W
            sem = sems[slot]
            pltpu.async_copy(src_hbm.at[pl.ds(off, W)], sbufs[slot], sem)
            pltpu.async_copy(dst_hbm.at[pl.ds(off, W)], dbufs[slot], sem)
            pltpu.async_copy(cT_hbm.at[pl.ds(c0 * E + off, W)],
                             cbufs[slot][0], sem)
            pltpu.async_copy(cT_hbm.at[pl.ds((c0 + 1) * E + off, W)],
                             cbufs[slot][1], sem)

        def drain(slot):
            sem = sems[slot]
            pltpu.make_async_copy(src_hbm.at[pl.ds(0, W)], sbufs[slot],
                                  sem).wait()
            pltpu.make_async_copy(dst_hbm.at[pl.ds(0, W)], dbufs[slot],
                                  sem).wait()
            pltpu.make_async_copy(cT_hbm.at[pl.ds(0, W)], cbufs[slot][0],
                                  sem).wait()
            pltpu.make_async_copy(cT_hbm.at[pl.ds(0, W)], cbufs[slot][1],
                                  sem).wait()

        def segcomb(x, eqs, is_add):
            for (shd, _), eq in zip(shifts, eqs):
                sv16[...] = x
                xs = plsc.load_gather(sv16, [shd])
                if is_add:
                    x = x + jnp.where(eq, xs, jnp.float32(0.0))
                else:
                    x = jnp.maximum(x, jnp.where(eq, xs, jnp.float32(NEG)))
            return x

        def process(slot, base_scalar):
            sb = sbufs[slot]
            db = dbufs[slot]

            @pl.loop(0, VPW)
            def _vec(v):
                base = v * 16
                dst16 = db[pl.ds(base, 16)]
                dstS, perm = plsc.sort_key_val(dst16, iota)
                sk16[...] = dstS
                eqs = []
                for shd, ged in shifts:
                    kd = plsc.load_gather(sk16, [shd])
                    eqs.append((kd == dstS) & ged)
                kp1 = plsc.load_gather(sk16, [shp1])
                last = (kp1 != dstS) | is15
                absp = perm + base
                for j in (0, 1):
                    srcS = plsc.load_gather(sb, [absp])
                    a = plsc.load_gather(acols[j], [srcS])
                    b = plsc.load_gather(bcols[j], [dstS])
                    cv = plsc.load_gather(cbufs[slot][j], [absp])
                    m = a + b + cv
                    sq = m * m
                    asum, asq, amx, amn = accs[j]
                    vs = segcomb(m, eqs, True)
                    cur = plsc.load_gather(asum, [dstS])
                    plsc.store_scatter(asum, [dstS], cur + vs, mask=last)
                    vq = segcomb(sq, eqs, True)
                    cur = plsc.load_gather(asq, [dstS])
                    plsc.store_scatter(asq, [dstS], cur + vq, mask=last)
                    vx = segcomb(m, eqs, False)
                    cur = plsc.load_gather(amx, [dstS])
                    plsc.store_scatter(amx, [dstS], jnp.maximum(cur, vx),
                                       mask=last)
                    vn = segcomb(-m, eqs, False)
                    cur = plsc.load_gather(amn, [dstS])
                    plsc.store_scatter(amn, [dstS], jnp.maximum(cur, vn),
                                       mask=last)

        for p in (0, 1):
            c0 = 2 * wid + 64 * p

            @pl.loop(0, N, step=16)
            def _init(i):
                z = jnp.zeros((16,), jnp.float32)
                ng = jnp.full((16,), NEG, jnp.float32)
                for j in (0, 1):
                    asum, asq, amx, amn = accs[j]
                    asum[pl.ds(i, 16)] = z
                    asq[pl.ds(i, 16)] = z
                    amx[pl.ds(i, 16)] = ng
                    amn[pl.ds(i, 16)] = ng

            for j in (0, 1):
                pltpu.async_copy(aT_hbm.at[pl.ds((c0 + j) * N, N)],
                                 acols[j], sems[0]).wait()
                pltpu.async_copy(bT_hbm.at[pl.ds((c0 + j) * N, N)],
                                 bcols[j], sems[0]).wait()

            issue(0, 0, c0)

            @pl.loop(0, NWIN, step=2)
            def _win(w):
                issue(1, w + 1, c0)
                drain(0)
                process(0, w * W)

                @pl.when(w + 2 < NWIN)
                def _nxt():
                    issue(0, w + 2, c0)

                drain(1)
                process(1, (w + 1) * W)

            for j in (0, 1):
                asum, asq, amx, amn = accs[j]
                sl = pl.ds((c0 + j) * N, N)
                pltpu.sync_copy(asum, osum.at[sl])
                pltpu.sync_copy(asq, osq.at[sl])
                pltpu.sync_copy(amx, omx.at[sl])
                pltpu.sync_copy(amn, omn.at[sl])

    outs = k(aT.reshape(-1), bT.reshape(-1), cT.reshape(-1), src, dst)
    return [o.reshape(D, N) for o in outs]


def kernel(edge_index, r, atom_features, distances, graph_ids, af_table,
           W_atom, b_atom, W_dist, b_dist, ln_g, ln_b, W_edge, b_edge,
           W_pre0, b_pre0, W_post0, b_post0, W_pre1, b_pre1, W_post1, b_post1,
           W_out, b_out):
    src = edge_index[0].astype(jnp.int32)
    dst = edge_index[1].astype(jnp.int32)

    h, a0T, b0T = _node_prep(atom_features.astype(jnp.int32), distances,
                             af_table, W_atom, b_atom, W_dist, b_dist,
                             ln_g, ln_b, W_pre0[:D], W_pre0[D:2 * D])
    c0T, c1T = _edge_prep(r, W_edge, b_edge, W_pre0[2 * D:], b_pre0,
                          W_pre1[2 * D:], b_pre1)
    degp = _deg_sc(dst)

    s, q, x, n = _edge_stage(a0T, b0T, c0T, src, dst)
    h1 = _post(h, s, q, x, n, degp, W_post0, b_post0)
    a1T, b1T = _abt(h1, W_pre1[:D], W_pre1[D:2 * D])
    s, q, x, n = _edge_stage(a1T, b1T, c1T, src, dst)
    h2 = _post(h1, s, q, x, n, degp, W_post1, b_post1)
    return _readout(h2, graph_ids.astype(jnp.int32), W_out, b_out)


# speedup vs baseline: 2.0006x; 2.0006x over previous
import dataclasses
import functools

import jax
import jax.numpy as jnp
import numpy as np
from jax import lax
from jax.experimental import pallas as pl
from jax.experimental.pallas import tpu as pltpu
from jax.experimental.pallas import tpu_sc as plsc

N = 10000
E = 160000
D = 128
EF = 40
MAXN = 12
NG = 100
NATOM = 100
AENC = 200
DELTA = float(np.log(MAXN + 1.0))
NEG = np.float32(-3.0e38)

EB = 6400
W = 800
NWIN = E // W
VPW = W // 16
CHUNK = E // 32


def _node_prep_body(atom_ref, dist_ref, af_ref, Wa_ref, ba_ref, Wd_ref,
                    bd_ref, g_ref, b_ref, Wp1_ref, Wp2_ref,
                    h_ref, aT_ref, bT_ref):
    atom = atom_ref[...]
    onehot = (lax.broadcasted_iota(jnp.int32, (N, NATOM), 1)
              == atom[:, None]).astype(jnp.float32)
    table2 = jnp.dot(af_ref[...], Wa_ref[...],
                     preferred_element_type=jnp.float32)
    hp = jnp.dot(onehot, table2, preferred_element_type=jnp.float32)
    hp = hp + ba_ref[...][None, :]
    hp = hp + jnp.dot(dist_ref[...], Wd_ref[...],
                      preferred_element_type=jnp.float32) + bd_ref[...][None, :]
    mu = jnp.mean(hp, axis=1, keepdims=True)
    va = jnp.mean((hp - mu) * (hp - mu), axis=1, keepdims=True)
    h = (hp - mu) * lax.rsqrt(va + 1e-5) * g_ref[...][None, :] + b_ref[...][None, :]
    h_ref[...] = h
    aT_ref[...] = lax.dot_general(Wp1_ref[...], h, (((0,), (1,)), ((), ())),
                                  preferred_element_type=jnp.float32)
    bT_ref[...] = lax.dot_general(Wp2_ref[...], h, (((0,), (1,)), ((), ())),
                                  preferred_element_type=jnp.float32)


def _node_prep(atom, dist, af_table, W_atom, b_atom, W_dist, b_dist,
               ln_g, ln_b, Wp1, Wp2):
    return pl.pallas_call(
        _node_prep_body,
        out_shape=[jax.ShapeDtypeStruct((N, D), jnp.float32),
                   jax.ShapeDtypeStruct((D, N), jnp.float32),
                   jax.ShapeDtypeStruct((D, N), jnp.float32)],
    )(atom, dist, af_table, W_atom, b_atom, W_dist, b_dist, ln_g, ln_b,
      Wp1, Wp2)


_CENTERS = np.linspace(0.0, 1.0, EF).astype(np.float32)
_GAMMA = np.float32(1.0 / (_CENTERS[1] - _CENTERS[0]) ** 2)


def _edge_prep_body(r_ref, We_ref, be_ref, Wp30_ref, bp0_ref, Wp31_ref,
                    bp1_ref, c0_ref, c1_ref):
    r = r_ref[...]
    d2 = jnp.sum(r * r, axis=0)
    invd = 1.0 / jnp.sqrt(d2)
    centers = (lax.broadcasted_iota(jnp.int32, (EF, 1), 0)
               .astype(jnp.float32) * np.float32(1.0 / (EF - 1)))
    gT = jnp.exp(-_GAMMA * (invd[None, :] - centers) ** 2)
    for (Wp3_ref, bp_ref, c_ref) in ((Wp30_ref, bp0_ref, c0_ref),
                                     (Wp31_ref, bp1_ref, c1_ref)):
        U = jnp.dot(We_ref[...], Wp3_ref[...],
                    preferred_element_type=jnp.float32)
        cst = jnp.dot(be_ref[...][None, :], Wp3_ref[...],
                      preferred_element_type=jnp.float32)[0] + bp_ref[...]
        c_ref[...] = lax.dot_general(U, gT, (((0,), (0,)), ((), ())),
                                     preferred_element_type=jnp.float32) \
            + cst[:, None]


def _edge_prep(r, W_edge, b_edge, Wp30, bp0, Wp31, bp1):
    nblk = E // EB
    return pl.pallas_call(
        _edge_prep_body,
        grid=(nblk,),
        in_specs=[
            pl.BlockSpec((3, EB), lambda i: (0, i)),
            pl.BlockSpec((EF, D), lambda i: (0, 0)),
            pl.BlockSpec((D,), lambda i: (0,)),
            pl.BlockSpec((D, D), lambda i: (0, 0)),
            pl.BlockSpec((D,), lambda i: (0,)),
            pl.BlockSpec((D, D), lambda i: (0, 0)),
            pl.BlockSpec((D,), lambda i: (0,)),
        ],
        out_specs=[pl.BlockSpec((D, EB), lambda i: (0, i)),
                   pl.BlockSpec((D, EB), lambda i: (0, i))],
        out_shape=[jax.ShapeDtypeStruct((D, E), jnp.float32),
                   jax.ShapeDtypeStruct((D, E), jnp.float32)],
    )(r.T, W_edge, b_edge, Wp30, bp0, Wp31, bp1)


def _post_body(h_ref, s_ref, q_ref, x_ref, n_ref, degp_ref, Wq_ref, bq_ref,
               out_ref):
    deg = degp_ref[0, :] + degp_ref[1, :]
    degc = jnp.maximum(deg, 1.0)
    inv = 1.0 / degc
    meanT = s_ref[...] * inv[None, :]
    msqT = q_ref[...] * inv[None, :]
    varT = jnp.maximum(msqT - meanT * meanT, 0.0)
    pos = (deg > 0.0)[None, :]
    mxT = jnp.where(pos, x_ref[...], 0.0)
    mnT = jnp.where(pos, -n_ref[...], 0.0)
    att = (DELTA / jnp.log(degc + 1.0))[:, None]
    Wq = Wq_ref[...]

    def dgT(xT, k):
        return lax.dot_general(xT, Wq[D * k:D * (k + 1)],
                               (((0,), (0,)), ((), ())),
                               preferred_element_type=jnp.float32)

    out = jnp.dot(h_ref[...], Wq[:D], preferred_element_type=jnp.float32)
    out = out + dgT(meanT, 1) + att * dgT(meanT, 5)
    out = out + dgT(varT, 2) + att * dgT(varT, 6)
    out = out + dgT(mnT, 3) + att * dgT(mnT, 7)
    out = out + dgT(mxT, 4) + att * dgT(mxT, 8)
    out_ref[...] = out + bq_ref[...][None, :]


def _post(h, s, q, x, n, degp, Wq, bq):
    return pl.pallas_call(
        _post_body,
        out_shape=jax.ShapeDtypeStruct((N, D), jnp.float32),
    )(h, s, q, x, n, degp, Wq, bq)


def _abt_body(h_ref, Wp1_ref, Wp2_ref, aT_ref, bT_ref):
    h = h_ref[...]
    aT_ref[...] = lax.dot_general(Wp1_ref[...], h, (((0,), (1,)), ((), ())),
                                  preferred_element_type=jnp.float32)
    bT_ref[...] = lax.dot_general(Wp2_ref[...], h, (((0,), (1,)), ((), ())),
                                  preferred_element_type=jnp.float32)


def _abt(h, Wp1, Wp2):
    return pl.pallas_call(
        _abt_body,
        out_shape=[jax.ShapeDtypeStruct((D, N), jnp.float32),
                   jax.ShapeDtypeStruct((D, N), jnp.float32)],
    )(h, Wp1, Wp2)


def _readout_body(h_ref, gid_ref, w_ref, b_ref, o_ref):
    h = h_ref[...]
    gid = gid_ref[...]
    seg = lax.broadcasted_iota(jnp.int32, (NG, N), 0)
    onehot = (seg == gid[None, :]).astype(jnp.float32)
    s = jnp.dot(onehot, h, preferred_element_type=jnp.float32)
    cnt = jnp.maximum(jnp.sum(onehot, axis=1, keepdims=True), 1.0)
    pooled = s / cnt
    o_ref[...] = jnp.dot(pooled, w_ref[...],
                         preferred_element_type=jnp.float32) + b_ref[...][None, :]


def _readout(h, gid, W_out, b_out):
    return pl.pallas_call(
        _readout_body,
        out_shape=jax.ShapeDtypeStruct((NG, 1), jnp.float32),
    )(h, gid, W_out, b_out)


def _deg_sc(dst):
    mesh = plsc.VectorSubcoreMesh(core_axis_name="c", subcore_axis_name="s")

    @functools.partial(
        pl.kernel, mesh=mesh,
        out_type=jax.ShapeDtypeStruct((2 * N,), jnp.float32),
        scratch_types=[
            pltpu.VMEM((CHUNK,), jnp.int32),
            pltpu.VMEM((CHUNK,), jnp.float32),
            pltpu.VMEM_SHARED((N,), jnp.float32),
            pltpu.SemaphoreType.DMA,
        ])
    def k(dst_hbm, deg_hbm, dstv, onesv, shared_deg, sem):
        c = lax.axis_index("c")
        s = lax.axis_index("s")

        @pl.loop(0, CHUNK, step=16)
        def _fill(i):
            onesv[pl.ds(i, 16)] = jnp.zeros((16,), jnp.float32)

        @pl.when(s == 0)
        def _zero():
            pltpu.sync_copy(onesv, shared_deg.at[pl.ds(0, CHUNK)])
            pltpu.sync_copy(onesv, shared_deg.at[pl.ds(CHUNK, CHUNK)])

        @pl.loop(0, CHUNK, step=16)
        def _fill1(i):
            onesv[pl.ds(i, 16)] = jnp.ones((16,), jnp.float32)

        plsc.subcore_barrier()
        base = (c * 16 + s) * CHUNK
        pltpu.async_copy(dst_hbm.at[pl.ds(base, CHUNK)], dstv, sem).wait()
        pltpu.sync_copy(onesv, shared_deg.at[dstv], add=True)
        plsc.subcore_barrier()

        @pl.when(s == 0)
        def _out():
            for half in (0, 1):
                pltpu.sync_copy(shared_deg.at[pl.ds(half * CHUNK, CHUNK)],
                                onesv)
                pltpu.sync_copy(onesv,
                                deg_hbm.at[pl.ds(c * N + half * CHUNK, CHUNK)])

    return k(dst).reshape(2, N)


_IOTA16 = np.arange(16, dtype=np.int32)


def _edge_stage(aT, bT, cT, src, dst):
    mesh = plsc.VectorSubcoreMesh(core_axis_name="c", subcore_axis_name="s")

    vm = pltpu.VMEM
    scratch = ([vm((N,), jnp.float32)] * 8
               + [vm((N,), jnp.float32)] * 4
               + [vm((W,), jnp.int32)] * 4
               + [vm((W,), jnp.float32)] * 4
               + [vm((16,), jnp.int32), vm((16,), jnp.float32)]
               + [pltpu.SemaphoreType.DMA, pltpu.SemaphoreType.DMA])

    cp = pltpu.CompilerParams()
    if "needs_layout_passes" in pltpu.CompilerParams.__dataclass_fields__:
        cp = dataclasses.replace(cp, needs_layout_passes=False)

    @functools.partial(
        pl.kernel, mesh=mesh,
        out_type=[jax.ShapeDtypeStruct((D * N,), jnp.float32)] * 4,
        scratch_types=scratch, compiler_params=cp)
    def k(aT_hbm, bT_hbm, cT_hbm, src_hbm, dst_hbm,
          osum, osq, omx, omn,
          as0, as1, aq0, aq1, ax0, ax1, an0, an1,
          ac0, ac1, bc0, bc1,
          sb0, sb1, db0, db1,
          cb00, cb01, cb10, cb11,
          sk16, sv16, sem0, sem1):
        cidx = lax.axis_index("c")
        sidx = lax.axis_index("s")
        wid = sidx * 2 + cidx

        iota = lax.iota(jnp.int32, 16)
        sh1 = jnp.maximum(iota - 1, 0)
        sh2 = jnp.maximum(iota - 2, 0)
        sh4 = jnp.maximum(iota - 4, 0)
        sh8 = jnp.maximum(iota - 8, 0)
        shp1 = jnp.minimum(iota + 1, 15)
        ge1 = iota >= 1
        ge2 = iota >= 2
        ge4 = iota >= 4
        ge8 = iota >= 8
        is15 = iota == 15
        shifts = ((sh1, ge1), (sh2, ge2), (sh4, ge4), (sh8, ge8))

        accs = ((as0, aq0, ax0, an0), (as1, aq1, ax1, an1))
        acols = (ac0, ac1)
        bcols = (bc0, bc1)
        sbufs = (sb0, sb1)
        dbufs = (db0, db1)
        cbufs = ((cb00, cb01), (cb10, cb11))
        sems = (sem0, sem1)

        def issue(slot, w, c0):
            off = w * W
            sem = sems[slot]
            pltpu.async_copy(src_hbm.at[pl.ds(off, W)], sbufs[slot], sem)
            pltpu.async_copy(dst_hbm.at[pl.ds(off, W)], dbufs[slot], sem)
            pltpu.async_copy(cT_hbm.at[pl.ds(c0 * E + off, W)],
                             cbufs[slot][0], sem)
            pltpu.async_copy(cT_hbm.at[pl.ds((c0 + 1) * E + off, W)],
                             cbufs[slot][1], sem)

        def drain(slot):
            sem = sems[slot]
            pltpu.make_async_copy(src_hbm.at[pl.ds(0, W)], sbufs[slot],
                                  sem).wait()
            pltpu.make_async_copy(dst_hbm.at[pl.ds(0, W)], dbufs[slot],
                                  sem).wait()
            pltpu.make_async_copy(cT_hbm.at[pl.ds(0, W)], cbufs[slot][0],
                                  sem).wait()
            pltpu.make_async_copy(cT_hbm.at[pl.ds(0, W)], cbufs[slot][1],
                                  sem).wait()

        def process(slot, base_scalar):
            sb = sbufs[slot]
            db = dbufs[slot]

            @pl.loop(0, VPW)
            def _vec(v):
                base = v * 16
                dst16 = db[pl.ds(base, 16)]
                src16 = sb[pl.ds(base, 16)]
                for j in (0, 1):
                    a = plsc.load_gather(acols[j], [src16])
                    b = plsc.load_gather(bcols[j], [dst16])
                    cv = cbufs[slot][j][pl.ds(base, 16)]
                    m = a + b + cv
                    sq = m * m
                    mn = -m
                    asum, asq, amx, amn = accs[j]
                    plsc.addupdate_scatter(asum, [dst16], m)
                    plsc.addupdate_scatter(asq, [dst16], sq)
                    curx = plsc.load_gather(amx, [dst16])
                    newx = jnp.maximum(curx, m)
                    plsc.store_scatter(amx, [dst16], newx)
                    chkx = plsc.load_gather(amx, [dst16])
                    fx = chkx < newx
                    curn = plsc.load_gather(amn, [dst16])
                    newn = jnp.maximum(curn, mn)
                    plsc.store_scatter(amn, [dst16], newn)
                    chkn = plsc.load_gather(amn, [dst16])
                    fn = chkn < newn

                    @pl.when(jnp.any(fx | fn))
                    def _retry():
                        def rbody(_, carry):
                            fx, fn = carry
                            cur = plsc.load_gather(amx, [dst16])
                            new = jnp.maximum(cur, m)
                            plsc.store_scatter(amx, [dst16], new, mask=fx)
                            chk = plsc.load_gather(amx, [dst16])
                            fx = fx & (chk < new)
                            cur = plsc.load_gather(amn, [dst16])
                            new = jnp.maximum(cur, mn)
                            plsc.store_scatter(amn, [dst16], new, mask=fn)
                            chk = plsc.load_gather(amn, [dst16])
                            fn = fn & (chk < new)
                            return fx, fn

                        lax.fori_loop(0, 16, rbody, (fx, fn))

        for p in (0, 1):
            c0 = 2 * wid + 64 * p

            @pl.loop(0, N, step=16)
            def _init(i):
                z = jnp.zeros((16,), jnp.float32)
                ng = jnp.full((16,), NEG, jnp.float32)
                for j in (0, 1):
                    asum, asq, amx, amn = accs[j]
                    asum[pl.ds(i, 16)] = z
                    asq[pl.ds(i, 16)] = z
                    amx[pl.ds(i, 16)] = ng
                    amn[pl.ds(i, 16)] = ng

            for j in (0, 1):
                pltpu.async_copy(aT_hbm.at[pl.ds((c0 + j) * N, N)],
                                 acols[j], sems[0]).wait()
                pltpu.async_copy(bT_hbm.at[pl.ds((c0 + j) * N, N)],
                                 bcols[j], sems[0]).wait()

            issue(0, 0, c0)

            @pl.loop(0, NWIN, step=2)
            def _win(w):
                issue(1, w + 1, c0)
                drain(0)
                process(0, w * W)

                @pl.when(w + 2 < NWIN)
                def _nxt():
                    issue(0, w + 2, c0)

                drain(1)
                process(1, (w + 1) * W)

            for j in (0, 1):
                asum, asq, amx, amn = accs[j]
                sl = pl.ds((c0 + j) * N, N)
                pltpu.sync_copy(asum, osum.at[sl])
                pltpu.sync_copy(asq, osq.at[sl])
                pltpu.sync_copy(amx, omx.at[sl])
                pltpu.sync_copy(amn, omn.at[sl])

    outs = k(aT.reshape(-1), bT.reshape(-1), cT.reshape(-1), src, dst)
    return [o.reshape(D, N) for o in outs]


def kernel(edge_index, r, atom_features, distances, graph_ids, af_table,
           W_atom, b_atom, W_dist, b_dist, ln_g, ln_b, W_edge, b_edge,
           W_pre0, b_pre0, W_post0, b_post0, W_pre1, b_pre1, W_post1, b_post1,
           W_out, b_out):
    src = edge_index[0].astype(jnp.int32)
    dst = edge_index[1].astype(jnp.int32)

    h, a0T, b0T = _node_prep(atom_features.astype(jnp.int32), distances,
                             af_table, W_atom, b_atom, W_dist, b_dist,
                             ln_g, ln_b, W_pre0[:D], W_pre0[D:2 * D])
    c0T, c1T = _edge_prep(r, W_edge, b_edge, W_pre0[2 * D:], b_pre0,
                          W_pre1[2 * D:], b_pre1)
    degp = _deg_sc(dst)

    s, q, x, n = _edge_stage(a0T, b0T, c0T, src, dst)
    h1 = _post(h, s, q, x, n, degp, W_post0, b_post0)
    a1T, b1T = _abt(h1, W_pre1[:D], W_pre1[D:2 * D])
    s, q, x, n = _edge_stage(a1T, b1T, c1T, src, dst)
    h2 = _post(h1, s, q, x, n, degp, W_post1, b_post1)
    return _readout(h2, graph_ids.astype(jnp.int32), W_out, b_out)


# speedup vs baseline: 2.9012x; 1.4501x over previous
import dataclasses
import functools

import jax
import jax.numpy as jnp
import numpy as np
from jax import lax
from jax.experimental import pallas as pl
from jax.experimental.pallas import tpu as pltpu
from jax.experimental.pallas import tpu_sc as plsc

N = 10000
E = 160000
D = 128
EF = 40
MAXN = 12
NG = 100
NATOM = 100
AENC = 200
DELTA = float(np.log(MAXN + 1.0))
NEG = np.float32(-3.0e38)

EB = 6400
W = 800
NWIN = E // W
VPW = W // 16
CHUNK = E // 32


def _node_prep_body(atom_ref, dist_ref, af_ref, Wa_ref, ba_ref, Wd_ref,
                    bd_ref, g_ref, b_ref, Wp1_ref, Wp2_ref,
                    h_ref, aT_ref, bT_ref):
    atom = atom_ref[...]
    onehot = (lax.broadcasted_iota(jnp.int32, (N, NATOM), 1)
              == atom[:, None]).astype(jnp.float32)
    table2 = jnp.dot(af_ref[...], Wa_ref[...],
                     preferred_element_type=jnp.float32)
    hp = jnp.dot(onehot, table2, preferred_element_type=jnp.float32)
    hp = hp + ba_ref[...][None, :]
    hp = hp + jnp.dot(dist_ref[...], Wd_ref[...],
                      preferred_element_type=jnp.float32) + bd_ref[...][None, :]
    mu = jnp.mean(hp, axis=1, keepdims=True)
    va = jnp.mean((hp - mu) * (hp - mu), axis=1, keepdims=True)
    h = (hp - mu) * lax.rsqrt(va + 1e-5) * g_ref[...][None, :] + b_ref[...][None, :]
    h_ref[...] = h
    aT_ref[...] = lax.dot_general(Wp1_ref[...], h, (((0,), (1,)), ((), ())),
                                  preferred_element_type=jnp.float32)
    bT_ref[...] = lax.dot_general(Wp2_ref[...], h, (((0,), (1,)), ((), ())),
                                  preferred_element_type=jnp.float32)


def _node_prep(atom, dist, af_table, W_atom, b_atom, W_dist, b_dist,
               ln_g, ln_b, Wp1, Wp2):
    return pl.pallas_call(
        _node_prep_body,
        out_shape=[jax.ShapeDtypeStruct((N, D), jnp.float32),
                   jax.ShapeDtypeStruct((D, N), jnp.float32),
                   jax.ShapeDtypeStruct((D, N), jnp.float32)],
    )(atom, dist, af_table, W_atom, b_atom, W_dist, b_dist, ln_g, ln_b,
      Wp1, Wp2)


_CENTERS = np.linspace(0.0, 1.0, EF).astype(np.float32)
_GAMMA = np.float32(1.0 / (_CENTERS[1] - _CENTERS[0]) ** 2)


def _edge_prep_body(r_ref, We_ref, be_ref, Wp30_ref, bp0_ref, Wp31_ref,
                    bp1_ref, c0_ref, c1_ref):
    r = r_ref[...]
    d2 = jnp.sum(r * r, axis=0)
    invd = 1.0 / jnp.sqrt(d2)
    centers = (lax.broadcasted_iota(jnp.int32, (EF, 1), 0)
               .astype(jnp.float32) * np.float32(1.0 / (EF - 1)))
    gT = jnp.exp(-_GAMMA * (invd[None, :] - centers) ** 2)
    for (Wp3_ref, bp_ref, c_ref) in ((Wp30_ref, bp0_ref, c0_ref),
                                     (Wp31_ref, bp1_ref, c1_ref)):
        U = jnp.dot(We_ref[...], Wp3_ref[...],
                    preferred_element_type=jnp.float32)
        cst = jnp.dot(be_ref[...][None, :], Wp3_ref[...],
                      preferred_element_type=jnp.float32)[0] + bp_ref[...]
        c_ref[...] = lax.dot_general(U, gT, (((0,), (0,)), ((), ())),
                                     preferred_element_type=jnp.float32) \
            + cst[:, None]


def _edge_prep(r, W_edge, b_edge, Wp30, bp0, Wp31, bp1):
    nblk = E // EB
    return pl.pallas_call(
        _edge_prep_body,
        grid=(nblk,),
        in_specs=[
            pl.BlockSpec((3, EB), lambda i: (0, i)),
            pl.BlockSpec((EF, D), lambda i: (0, 0)),
            pl.BlockSpec((D,), lambda i: (0,)),
            pl.BlockSpec((D, D), lambda i: (0, 0)),
            pl.BlockSpec((D,), lambda i: (0,)),
            pl.BlockSpec((D, D), lambda i: (0, 0)),
            pl.BlockSpec((D,), lambda i: (0,)),
        ],
        out_specs=[pl.BlockSpec((D, EB), lambda i: (0, i)),
                   pl.BlockSpec((D, EB), lambda i: (0, i))],
        out_shape=[jax.ShapeDtypeStruct((D, E), jnp.float32),
                   jax.ShapeDtypeStruct((D, E), jnp.float32)],
    )(r.T, W_edge, b_edge, Wp30, bp0, Wp31, bp1)


def _post_body(h_ref, s_ref, q_ref, x_ref, n_ref, degp_ref, Wq_ref, bq_ref,
               out_ref):
    deg = degp_ref[0, :] + degp_ref[1, :]
    degc = jnp.maximum(deg, 1.0)
    inv = 1.0 / degc
    meanT = s_ref[...] * inv[None, :]
    msqT = q_ref[...] * inv[None, :]
    varT = jnp.maximum(msqT - meanT * meanT, 0.0)
    pos = (deg > 0.0)[None, :]
    mxT = jnp.where(pos, x_ref[...], 0.0)
    mnT = jnp.where(pos, -n_ref[...], 0.0)
    att = (DELTA / jnp.log(degc + 1.0))[:, None]
    Wq = Wq_ref[...]

    def dgT(xT, k):
        return lax.dot_general(xT, Wq[D * k:D * (k + 1)],
                               (((0,), (0,)), ((), ())),
                               preferred_element_type=jnp.float32)

    out = jnp.dot(h_ref[...], Wq[:D], preferred_element_type=jnp.float32)
    out = out + dgT(meanT, 1) + att * dgT(meanT, 5)
    out = out + dgT(varT, 2) + att * dgT(varT, 6)
    out = out + dgT(mnT, 3) + att * dgT(mnT, 7)
    out = out + dgT(mxT, 4) + att * dgT(mxT, 8)
    out_ref[...] = out + bq_ref[...][None, :]


def _post(h, s, q, x, n, degp, Wq, bq):
    return pl.pallas_call(
        _post_body,
        out_shape=jax.ShapeDtypeStruct((N, D), jnp.float32),
    )(h, s, q, x, n, degp, Wq, bq)


def _abt_body(h_ref, Wp1_ref, Wp2_ref, aT_ref, bT_ref):
    h = h_ref[...]
    aT_ref[...] = lax.dot_general(Wp1_ref[...], h, (((0,), (1,)), ((), ())),
                                  preferred_element_type=jnp.float32)
    bT_ref[...] = lax.dot_general(Wp2_ref[...], h, (((0,), (1,)), ((), ())),
                                  preferred_element_type=jnp.float32)


def _abt(h, Wp1, Wp2):
    return pl.pallas_call(
        _abt_body,
        out_shape=[jax.ShapeDtypeStruct((D, N), jnp.float32),
                   jax.ShapeDtypeStruct((D, N), jnp.float32)],
    )(h, Wp1, Wp2)


def _readout_body(h_ref, gid_ref, w_ref, b_ref, o_ref):
    h = h_ref[...]
    gid = gid_ref[...]
    seg = lax.broadcasted_iota(jnp.int32, (NG, N), 0)
    onehot = (seg == gid[None, :]).astype(jnp.float32)
    s = jnp.dot(onehot, h, preferred_element_type=jnp.float32)
    cnt = jnp.maximum(jnp.sum(onehot, axis=1, keepdims=True), 1.0)
    pooled = s / cnt
    o_ref[...] = jnp.dot(pooled, w_ref[...],
                         preferred_element_type=jnp.float32) + b_ref[...][None, :]


def _readout(h, gid, W_out, b_out):
    return pl.pallas_call(
        _readout_body,
        out_shape=jax.ShapeDtypeStruct((NG, 1), jnp.float32),
    )(h, gid, W_out, b_out)


def _deg_sc(dst):
    mesh = plsc.VectorSubcoreMesh(core_axis_name="c", subcore_axis_name="s")

    @functools.partial(
        pl.kernel, mesh=mesh,
        out_type=jax.ShapeDtypeStruct((2 * N,), jnp.float32),
        scratch_types=[
            pltpu.VMEM((CHUNK,), jnp.int32),
            pltpu.VMEM((CHUNK,), jnp.float32),
            pltpu.VMEM_SHARED((N,), jnp.float32),
            pltpu.SemaphoreType.DMA,
        ])
    def k(dst_hbm, deg_hbm, dstv, onesv, shared_deg, sem):
        c = lax.axis_index("c")
        s = lax.axis_index("s")

        @pl.loop(0, CHUNK, step=16)
        def _fill(i):
            onesv[pl.ds(i, 16)] = jnp.zeros((16,), jnp.float32)

        @pl.when(s == 0)
        def _zero():
            pltpu.sync_copy(onesv, shared_deg.at[pl.ds(0, CHUNK)])
            pltpu.sync_copy(onesv, shared_deg.at[pl.ds(CHUNK, CHUNK)])

        @pl.loop(0, CHUNK, step=16)
        def _fill1(i):
            onesv[pl.ds(i, 16)] = jnp.ones((16,), jnp.float32)

        plsc.subcore_barrier()
        base = (c * 16 + s) * CHUNK
        pltpu.async_copy(dst_hbm.at[pl.ds(base, CHUNK)], dstv, sem).wait()
        pltpu.sync_copy(onesv, shared_deg.at[dstv], add=True)
        plsc.subcore_barrier()

        @pl.when(s == 0)
        def _out():
            for half in (0, 1):
                pltpu.sync_copy(shared_deg.at[pl.ds(half * CHUNK, CHUNK)],
                                onesv)
                pltpu.sync_copy(onesv,
                                deg_hbm.at[pl.ds(c * N + half * CHUNK, CHUNK)])

    return k(dst).reshape(2, N)


_IOTA16 = np.arange(16, dtype=np.int32)


def _edge_stage(aT, bT, cT, src, dst):
    mesh = plsc.VectorSubcoreMesh(core_axis_name="c", subcore_axis_name="s")

    vm = pltpu.VMEM
    scratch = ([vm((N,), jnp.float32)] * 8
               + [vm((N,), jnp.float32)] * 4
               + [vm((W,), jnp.int32)] * 4
               + [vm((W,), jnp.float32)] * 4
               + [vm((16,), jnp.int32), vm((16,), jnp.float32)]
               + [pltpu.SemaphoreType.DMA, pltpu.SemaphoreType.DMA])

    cp = pltpu.CompilerParams()
    if "needs_layout_passes" in pltpu.CompilerParams.__dataclass_fields__:
        cp = dataclasses.replace(cp, needs_layout_passes=False)

    @functools.partial(
        pl.kernel, mesh=mesh,
        out_type=[jax.ShapeDtypeStruct((D * N,), jnp.float32)] * 4,
        scratch_types=scratch, compiler_params=cp)
    def k(aT_hbm, bT_hbm, cT_hbm, src_hbm, dst_hbm,
          osum, osq, omx, omn,
          as0, as1, aq0, aq1, ax0, ax1, an0, an1,
          ac0, ac1, bc0, bc1,
          sb0, sb1, db0, db1,
          cb00, cb01, cb10, cb11,
          sk16, sv16, sem0, sem1):
        cidx = lax.axis_index("c")
        sidx = lax.axis_index("s")
        wid = sidx * 2 + cidx

        iota = lax.iota(jnp.int32, 16)
        sh1 = jnp.maximum(iota - 1, 0)
        sh2 = jnp.maximum(iota - 2, 0)
        sh4 = jnp.maximum(iota - 4, 0)
        sh8 = jnp.maximum(iota - 8, 0)
        shp1 = jnp.minimum(iota + 1, 15)
        ge1 = iota >= 1
        ge2 = iota >= 2
        ge4 = iota >= 4
        ge8 = iota >= 8
        is15 = iota == 15
        shifts = ((sh1, ge1), (sh2, ge2), (sh4, ge4), (sh8, ge8))

        accs = ((as0, aq0, ax0, an0), (as1, aq1, ax1, an1))
        acols = (ac0, ac1)
        bcols = (bc0, bc1)
        sbufs = (sb0, sb1)
        dbufs = (db0, db1)
        cbufs = ((cb00, cb01), (cb10, cb11))
        sems = (sem0, sem1)

        def issue(slot, w, c0):
            off = w * W
            sem = sems[slot]
            pltpu.async_copy(src_hbm.at[pl.ds(off, W)], sbufs[slot], sem)
            pltpu.async_copy(dst_hbm.at[pl.ds(off, W)], dbufs[slot], sem)
            pltpu.async_copy(cT_hbm.at[pl.ds(c0 * E + off, W)],
                             cbufs[slot][0], sem)
            pltpu.async_copy(cT_hbm.at[pl.ds((c0 + 1) * E + off, W)],
                             cbufs[slot][1], sem)

        def drain(slot):
            sem = sems[slot]
            pltpu.make_async_copy(src_hbm.at[pl.ds(0, W)], sbufs[slot],
                                  sem).wait()
            pltpu.make_async_copy(dst_hbm.at[pl.ds(0, W)], dbufs[slot],
                                  sem).wait()
            pltpu.make_async_copy(cT_hbm.at[pl.ds(0, W)], cbufs[slot][0],
                                  sem).wait()
            pltpu.make_async_copy(cT_hbm.at[pl.ds(0, W)], cbufs[slot][1],
                                  sem).wait()

        def process(slot, base_scalar):
            sb = sbufs[slot]
            db = dbufs[slot]

            @pl.loop(0, VPW, step=2)
            def _vec(v):
                work = []
                for u in (0, 1):
                    b2 = (v + u) * 16
                    dst16 = db[pl.ds(b2, 16)]
                    src16 = sb[pl.ds(b2, 16)]
                    for j in (0, 1):
                        a = plsc.load_gather(acols[j], [src16])
                        b = plsc.load_gather(bcols[j], [dst16])
                        cv = cbufs[slot][j][pl.ds(b2, 16)]
                        m = a + b + cv
                        sq = m * m
                        asum, asq, _, _ = accs[j]
                        plsc.addupdate_scatter(asum, [dst16], m)
                        plsc.addupdate_scatter(asq, [dst16], sq)
                        work.append((j, dst16, m, -m))
                news = []
                for j, dst16, m, mn in work:
                    _, _, amx, amn = accs[j]
                    newx = jnp.maximum(plsc.load_gather(amx, [dst16]), m)
                    plsc.store_scatter(amx, [dst16], newx)
                    newn = jnp.maximum(plsc.load_gather(amn, [dst16]), mn)
                    plsc.store_scatter(amn, [dst16], newn)
                    news.append((newx, newn))
                fails = []
                anyf = None
                for (j, dst16, m, mn), (newx, newn) in zip(work, news):
                    _, _, amx, amn = accs[j]
                    fx = plsc.load_gather(amx, [dst16]) < newx
                    fn = plsc.load_gather(amn, [dst16]) < newn
                    fails.append((fx, fn))
                    f = fx | fn
                    anyf = f if anyf is None else (anyf | f)

                @pl.when(jnp.any(anyf))
                def _retry():
                    def rbody(_, carry):
                        out = []
                        for (j, dst16, m, mn), (fx, fn) in zip(work, carry):
                            _, _, amx, amn = accs[j]
                            new = jnp.maximum(plsc.load_gather(amx, [dst16]),
                                              m)
                            plsc.store_scatter(amx, [dst16], new, mask=fx)
                            fx = fx & (plsc.load_gather(amx, [dst16]) < new)
                            new = jnp.maximum(plsc.load_gather(amn, [dst16]),
                                              mn)
                            plsc.store_scatter(amn, [dst16], new, mask=fn)
                            fn = fn & (plsc.load_gather(amn, [dst16]) < new)
                            out.append((fx, fn))
                        return tuple(out)

                    lax.fori_loop(0, 16, rbody, tuple(fails))

        for p in (0, 1):
            c0 = 2 * wid + 64 * p

            @pl.loop(0, N, step=16)
            def _init(i):
                z = jnp.zeros((16,), jnp.float32)
                ng = jnp.full((16,), NEG, jnp.float32)
                for j in (0, 1):
                    asum, asq, amx, amn = accs[j]
                    asum[pl.ds(i, 16)] = z
                    asq[pl.ds(i, 16)] = z
                    amx[pl.ds(i, 16)] = ng
                    amn[pl.ds(i, 16)] = ng

            for j in (0, 1):
                pltpu.async_copy(aT_hbm.at[pl.ds((c0 + j) * N, N)],
                                 acols[j], sems[0]).wait()
                pltpu.async_copy(bT_hbm.at[pl.ds((c0 + j) * N, N)],
                                 bcols[j], sems[0]).wait()

            issue(0, 0, c0)

            @pl.loop(0, NWIN, step=2)
            def _win(w):
                issue(1, w + 1, c0)
                drain(0)
                process(0, w * W)

                @pl.when(w + 2 < NWIN)
                def _nxt():
                    issue(0, w + 2, c0)

                drain(1)
                process(1, (w + 1) * W)

            for j in (0, 1):
                asum, asq, amx, amn = accs[j]
                sl = pl.ds((c0 + j) * N, N)
                pltpu.sync_copy(asum, osum.at[sl])
                pltpu.sync_copy(asq, osq.at[sl])
                pltpu.sync_copy(amx, omx.at[sl])
                pltpu.sync_copy(amn, omn.at[sl])

    outs = k(aT.reshape(-1), bT.reshape(-1), cT.reshape(-1), src, dst)
    return [o.reshape(D, N) for o in outs]


def kernel(edge_index, r, atom_features, distances, graph_ids, af_table,
           W_atom, b_atom, W_dist, b_dist, ln_g, ln_b, W_edge, b_edge,
           W_pre0, b_pre0, W_post0, b_post0, W_pre1, b_pre1, W_post1, b_post1,
           W_out, b_out):
    src = edge_index[0].astype(jnp.int32)
    dst = edge_index[1].astype(jnp.int32)

    h, a0T, b0T = _node_prep(atom_features.astype(jnp.int32), distances,
                             af_table, W_atom, b_atom, W_dist, b_dist,
                             ln_g, ln_b, W_pre0[:D], W_pre0[D:2 * D])
    c0T, c1T = _edge_prep(r, W_edge, b_edge, W_pre0[2 * D:], b_pre0,
                          W_pre1[2 * D:], b_pre1)
    degp = _deg_sc(dst)

    s, q, x, n = _edge_stage(a0T, b0T, c0T, src, dst)
    h1 = _post(h, s, q, x, n, degp, W_post0, b_post0)
    a1T, b1T = _abt(h1, W_pre1[:D], W_pre1[D:2 * D])
    s, q, x, n = _edge_stage(a1T, b1T, c1T, src, dst)
    h2 = _post(h1, s, q, x, n, degp, W_post1, b_post1)
    return _readout(h2, graph_ids.astype(jnp.int32), W_out, b_out)


# speedup vs baseline: 2.9839x; 1.0285x over previous
import dataclasses
import functools

import jax
import jax.numpy as jnp
import numpy as np
from jax import lax
from jax.experimental import pallas as pl
from jax.experimental.pallas import tpu as pltpu
from jax.experimental.pallas import tpu_sc as plsc

N = 10000
E = 160000
D = 128
EF = 40
MAXN = 12
NG = 100
NATOM = 100
AENC = 200
DELTA = float(np.log(MAXN + 1.0))
NEG = np.float32(-3.0e38)

EB = 6400
W = 640
NWIN = E // W
VPW = W // 16
CHUNK = E // 32


def _node_prep_body(atom_ref, dist_ref, af_ref, Wa_ref, ba_ref, Wd_ref,
                    bd_ref, g_ref, b_ref, Wp1_ref, Wp2_ref,
                    h_ref, aT_ref, bT_ref):
    atom = atom_ref[...]
    onehot = (lax.broadcasted_iota(jnp.int32, (N, NATOM), 1)
              == atom[:, None]).astype(jnp.float32)
    table2 = jnp.dot(af_ref[...], Wa_ref[...],
                     preferred_element_type=jnp.float32)
    hp = jnp.dot(onehot, table2, preferred_element_type=jnp.float32)
    hp = hp + ba_ref[...][None, :]
    hp = hp + jnp.dot(dist_ref[...], Wd_ref[...],
                      preferred_element_type=jnp.float32) + bd_ref[...][None, :]
    mu = jnp.mean(hp, axis=1, keepdims=True)
    va = jnp.mean((hp - mu) * (hp - mu), axis=1, keepdims=True)
    h = (hp - mu) * lax.rsqrt(va + 1e-5) * g_ref[...][None, :] + b_ref[...][None, :]
    h_ref[...] = h
    aT_ref[...] = lax.dot_general(Wp1_ref[...], h, (((0,), (1,)), ((), ())),
                                  preferred_element_type=jnp.float32)
    bT_ref[...] = lax.dot_general(Wp2_ref[...], h, (((0,), (1,)), ((), ())),
                                  preferred_element_type=jnp.float32)


def _node_prep(atom, dist, af_table, W_atom, b_atom, W_dist, b_dist,
               ln_g, ln_b, Wp1, Wp2):
    return pl.pallas_call(
        _node_prep_body,
        out_shape=[jax.ShapeDtypeStruct((N, D), jnp.float32),
                   jax.ShapeDtypeStruct((D, N), jnp.float32),
                   jax.ShapeDtypeStruct((D, N), jnp.float32)],
    )(atom, dist, af_table, W_atom, b_atom, W_dist, b_dist, ln_g, ln_b,
      Wp1, Wp2)


_CENTERS = np.linspace(0.0, 1.0, EF).astype(np.float32)
_GAMMA = np.float32(1.0 / (_CENTERS[1] - _CENTERS[0]) ** 2)


def _edge_prep_body(r_ref, We_ref, be_ref, Wp30_ref, bp0_ref, Wp31_ref,
                    bp1_ref, c0_ref, c1_ref):
    r = r_ref[...]
    d2 = jnp.sum(r * r, axis=0)
    invd = 1.0 / jnp.sqrt(d2)
    centers = (lax.broadcasted_iota(jnp.int32, (EF, 1), 0)
               .astype(jnp.float32) * np.float32(1.0 / (EF - 1)))
    gT = jnp.exp(-_GAMMA * (invd[None, :] - centers) ** 2)
    for (Wp3_ref, bp_ref, c_ref) in ((Wp30_ref, bp0_ref, c0_ref),
                                     (Wp31_ref, bp1_ref, c1_ref)):
        U = jnp.dot(We_ref[...], Wp3_ref[...],
                    preferred_element_type=jnp.float32)
        cst = jnp.dot(be_ref[...][None, :], Wp3_ref[...],
                      preferred_element_type=jnp.float32)[0] + bp_ref[...]
        c_ref[...] = lax.dot_general(U, gT, (((0,), (0,)), ((), ())),
                                     preferred_element_type=jnp.float32) \
            + cst[:, None]


def _edge_prep(r, W_edge, b_edge, Wp30, bp0, Wp31, bp1):
    nblk = E // EB
    return pl.pallas_call(
        _edge_prep_body,
        grid=(nblk,),
        in_specs=[
            pl.BlockSpec((3, EB), lambda i: (0, i)),
            pl.BlockSpec((EF, D), lambda i: (0, 0)),
            pl.BlockSpec((D,), lambda i: (0,)),
            pl.BlockSpec((D, D), lambda i: (0, 0)),
            pl.BlockSpec((D,), lambda i: (0,)),
            pl.BlockSpec((D, D), lambda i: (0, 0)),
            pl.BlockSpec((D,), lambda i: (0,)),
        ],
        out_specs=[pl.BlockSpec((D, EB), lambda i: (0, i)),
                   pl.BlockSpec((D, EB), lambda i: (0, i))],
        out_shape=[jax.ShapeDtypeStruct((D, E), jnp.float32),
                   jax.ShapeDtypeStruct((D, E), jnp.float32)],
    )(r.T, W_edge, b_edge, Wp30, bp0, Wp31, bp1)


def _post_body(h_ref, s_ref, q_ref, x_ref, n_ref, degp_ref, Wq_ref, bq_ref,
               out_ref):
    deg = degp_ref[0, :] + degp_ref[1, :]
    degc = jnp.maximum(deg, 1.0)
    inv = 1.0 / degc
    meanT = s_ref[...] * inv[None, :]
    msqT = q_ref[...] * inv[None, :]
    varT = jnp.maximum(msqT - meanT * meanT, 0.0)
    pos = (deg > 0.0)[None, :]
    mxT = jnp.where(pos, x_ref[...], 0.0)
    mnT = jnp.where(pos, -n_ref[...], 0.0)
    att = (DELTA / jnp.log(degc + 1.0))[:, None]
    Wq = Wq_ref[...]

    def dgT(xT, k):
        return lax.dot_general(xT, Wq[D * k:D * (k + 1)],
                               (((0,), (0,)), ((), ())),
                               preferred_element_type=jnp.float32)

    out = jnp.dot(h_ref[...], Wq[:D], preferred_element_type=jnp.float32)
    out = out + dgT(meanT, 1) + att * dgT(meanT, 5)
    out = out + dgT(varT, 2) + att * dgT(varT, 6)
    out = out + dgT(mnT, 3) + att * dgT(mnT, 7)
    out = out + dgT(mxT, 4) + att * dgT(mxT, 8)
    out_ref[...] = out + bq_ref[...][None, :]


def _post(h, s, q, x, n, degp, Wq, bq):
    return pl.pallas_call(
        _post_body,
        out_shape=jax.ShapeDtypeStruct((N, D), jnp.float32),
    )(h, s, q, x, n, degp, Wq, bq)


def _abt_body(h_ref, Wp1_ref, Wp2_ref, aT_ref, bT_ref):
    h = h_ref[...]
    aT_ref[...] = lax.dot_general(Wp1_ref[...], h, (((0,), (1,)), ((), ())),
                                  preferred_element_type=jnp.float32)
    bT_ref[...] = lax.dot_general(Wp2_ref[...], h, (((0,), (1,)), ((), ())),
                                  preferred_element_type=jnp.float32)


def _abt(h, Wp1, Wp2):
    return pl.pallas_call(
        _abt_body,
        out_shape=[jax.ShapeDtypeStruct((D, N), jnp.float32),
                   jax.ShapeDtypeStruct((D, N), jnp.float32)],
    )(h, Wp1, Wp2)


def _readout_body(h_ref, gid_ref, w_ref, b_ref, o_ref):
    h = h_ref[...]
    gid = gid_ref[...]
    seg = lax.broadcasted_iota(jnp.int32, (NG, N), 0)
    onehot = (seg == gid[None, :]).astype(jnp.float32)
    s = jnp.dot(onehot, h, preferred_element_type=jnp.float32)
    cnt = jnp.maximum(jnp.sum(onehot, axis=1, keepdims=True), 1.0)
    pooled = s / cnt
    o_ref[...] = jnp.dot(pooled, w_ref[...],
                         preferred_element_type=jnp.float32) + b_ref[...][None, :]


def _readout(h, gid, W_out, b_out):
    return pl.pallas_call(
        _readout_body,
        out_shape=jax.ShapeDtypeStruct((NG, 1), jnp.float32),
    )(h, gid, W_out, b_out)


def _deg_sc(dst):
    mesh = plsc.VectorSubcoreMesh(core_axis_name="c", subcore_axis_name="s")

    @functools.partial(
        pl.kernel, mesh=mesh,
        out_type=jax.ShapeDtypeStruct((2 * N,), jnp.float32),
        scratch_types=[
            pltpu.VMEM((CHUNK,), jnp.int32),
            pltpu.VMEM((CHUNK,), jnp.float32),
            pltpu.VMEM_SHARED((N,), jnp.float32),
            pltpu.SemaphoreType.DMA,
        ])
    def k(dst_hbm, deg_hbm, dstv, onesv, shared_deg, sem):
        c = lax.axis_index("c")
        s = lax.axis_index("s")

        @pl.loop(0, CHUNK, step=16)
        def _fill(i):
            onesv[pl.ds(i, 16)] = jnp.zeros((16,), jnp.float32)

        @pl.when(s == 0)
        def _zero():
            pltpu.sync_copy(onesv, shared_deg.at[pl.ds(0, CHUNK)])
            pltpu.sync_copy(onesv, shared_deg.at[pl.ds(CHUNK, CHUNK)])

        @pl.loop(0, CHUNK, step=16)
        def _fill1(i):
            onesv[pl.ds(i, 16)] = jnp.ones((16,), jnp.float32)

        plsc.subcore_barrier()
        base = (c * 16 + s) * CHUNK
        pltpu.async_copy(dst_hbm.at[pl.ds(base, CHUNK)], dstv, sem).wait()
        pltpu.sync_copy(onesv, shared_deg.at[dstv], add=True)
        plsc.subcore_barrier()

        @pl.when(s == 0)
        def _out():
            for half in (0, 1):
                pltpu.sync_copy(shared_deg.at[pl.ds(half * CHUNK, CHUNK)],
                                onesv)
                pltpu.sync_copy(onesv,
                                deg_hbm.at[pl.ds(c * N + half * CHUNK, CHUNK)])

    return k(dst).reshape(2, N)


_IOTA16 = np.arange(16, dtype=np.int32)


def _edge_stage(aT, bT, cT, src, dst):
    mesh = plsc.VectorSubcoreMesh(core_axis_name="c", subcore_axis_name="s")

    vm = pltpu.VMEM
    scratch = ([vm((N,), jnp.float32)] * 8
               + [vm((N,), jnp.float32)] * 4
               + [vm((W,), jnp.int32)] * 4
               + [vm((W,), jnp.float32)] * 4
               + [vm((16,), jnp.int32), vm((16,), jnp.float32)]
               + [pltpu.SemaphoreType.DMA, pltpu.SemaphoreType.DMA])

    cp = pltpu.CompilerParams()
    if "needs_layout_passes" in pltpu.CompilerParams.__dataclass_fields__:
        cp = dataclasses.replace(cp, needs_layout_passes=False)

    @functools.partial(
        pl.kernel, mesh=mesh,
        out_type=[jax.ShapeDtypeStruct((D * N,), jnp.float32)] * 4,
        scratch_types=scratch, compiler_params=cp)
    def k(aT_hbm, bT_hbm, cT_hbm, src_hbm, dst_hbm,
          osum, osq, omx, omn,
          as0, as1, aq0, aq1, ax0, ax1, an0, an1,
          ac0, ac1, bc0, bc1,
          sb0, sb1, db0, db1,
          cb00, cb01, cb10, cb11,
          sk16, sv16, sem0, sem1):
        cidx = lax.axis_index("c")
        sidx = lax.axis_index("s")
        wid = sidx * 2 + cidx

        iota = lax.iota(jnp.int32, 16)
        sh1 = jnp.maximum(iota - 1, 0)
        sh2 = jnp.maximum(iota - 2, 0)
        sh4 = jnp.maximum(iota - 4, 0)
        sh8 = jnp.maximum(iota - 8, 0)
        shp1 = jnp.minimum(iota + 1, 15)
        ge1 = iota >= 1
        ge2 = iota >= 2
        ge4 = iota >= 4
        ge8 = iota >= 8
        is15 = iota == 15
        shifts = ((sh1, ge1), (sh2, ge2), (sh4, ge4), (sh8, ge8))

        accs = ((as0, aq0, ax0, an0), (as1, aq1, ax1, an1))
        acols = (ac0, ac1)
        bcols = (bc0, bc1)
        sbufs = (sb0, sb1)
        dbufs = (db0, db1)
        cbufs = ((cb00, cb01), (cb10, cb11))
        sems = (sem0, sem1)

        def issue(slot, w, c0):
            off = w * W
            sem = sems[slot]
            pltpu.async_copy(src_hbm.at[pl.ds(off, W)], sbufs[slot], sem)
            pltpu.async_copy(dst_hbm.at[pl.ds(off, W)], dbufs[slot], sem)
            pltpu.async_copy(cT_hbm.at[pl.ds(c0 * E + off, W)],
                             cbufs[slot][0], sem)
            pltpu.async_copy(cT_hbm.at[pl.ds((c0 + 1) * E + off, W)],
                             cbufs[slot][1], sem)

        def drain(slot):
            sem = sems[slot]
            pltpu.make_async_copy(src_hbm.at[pl.ds(0, W)], sbufs[slot],
                                  sem).wait()
            pltpu.make_async_copy(dst_hbm.at[pl.ds(0, W)], dbufs[slot],
                                  sem).wait()
            pltpu.make_async_copy(cT_hbm.at[pl.ds(0, W)], cbufs[slot][0],
                                  sem).wait()
            pltpu.make_async_copy(cT_hbm.at[pl.ds(0, W)], cbufs[slot][1],
                                  sem).wait()

        def process(slot, base_scalar):
            sb = sbufs[slot]
            db = dbufs[slot]

            @pl.loop(0, VPW, step=4)
            def _vec(v):
                work = []
                for u in (0, 1, 2, 3):
                    b2 = (v + u) * 16
                    dst16 = db[pl.ds(b2, 16)]
                    src16 = sb[pl.ds(b2, 16)]
                    for j in (0, 1):
                        a = plsc.load_gather(acols[j], [src16])
                        b = plsc.load_gather(bcols[j], [dst16])
                        cv = cbufs[slot][j][pl.ds(b2, 16)]
                        m = a + b + cv
                        sq = m * m
                        asum, asq, _, _ = accs[j]
                        plsc.addupdate_scatter(asum, [dst16], m)
                        plsc.addupdate_scatter(asq, [dst16], sq)
                        work.append((j, dst16, m, -m))
                news = []
                for j, dst16, m, mn in work:
                    _, _, amx, amn = accs[j]
                    newx = jnp.maximum(plsc.load_gather(amx, [dst16]), m)
                    plsc.store_scatter(amx, [dst16], newx)
                    newn = jnp.maximum(plsc.load_gather(amn, [dst16]), mn)
                    plsc.store_scatter(amn, [dst16], newn)
                    news.append((newx, newn))
                fails = []
                anyf = None
                for (j, dst16, m, mn), (newx, newn) in zip(work, news):
                    _, _, amx, amn = accs[j]
                    fx = plsc.load_gather(amx, [dst16]) < newx
                    fn = plsc.load_gather(amn, [dst16]) < newn
                    fails.append((fx, fn))
                    f = fx | fn
                    anyf = f if anyf is None else (anyf | f)

                @pl.when(jnp.any(anyf))
                def _retry():
                    def rbody(_, carry):
                        out = []
                        for (j, dst16, m, mn), (fx, fn) in zip(work, carry):
                            _, _, amx, amn = accs[j]
                            new = jnp.maximum(plsc.load_gather(amx, [dst16]),
                                              m)
                            plsc.store_scatter(amx, [dst16], new, mask=fx)
                            fx = fx & (plsc.load_gather(amx, [dst16]) < new)
                            new = jnp.maximum(plsc.load_gather(amn, [dst16]),
                                              mn)
                            plsc.store_scatter(amn, [dst16], new, mask=fn)
                            fn = fn & (plsc.load_gather(amn, [dst16]) < new)
                            out.append((fx, fn))
                        return tuple(out)

                    lax.fori_loop(0, 16, rbody, tuple(fails))

        for p in (0, 1):
            c0 = 2 * wid + 64 * p

            @pl.loop(0, N, step=16)
            def _init(i):
                z = jnp.zeros((16,), jnp.float32)
                ng = jnp.full((16,), NEG, jnp.float32)
                for j in (0, 1):
                    asum, asq, amx, amn = accs[j]
                    asum[pl.ds(i, 16)] = z
                    asq[pl.ds(i, 16)] = z
                    amx[pl.ds(i, 16)] = ng
                    amn[pl.ds(i, 16)] = ng

            for j in (0, 1):
                pltpu.async_copy(aT_hbm.at[pl.ds((c0 + j) * N, N)],
                                 acols[j], sems[0]).wait()
                pltpu.async_copy(bT_hbm.at[pl.ds((c0 + j) * N, N)],
                                 bcols[j], sems[0]).wait()

            issue(0, 0, c0)

            @pl.loop(0, NWIN, step=2)
            def _win(w):
                issue(1, w + 1, c0)
                drain(0)
                process(0, w * W)

                @pl.when(w + 2 < NWIN)
                def _nxt():
                    issue(0, w + 2, c0)

                drain(1)
                process(1, (w + 1) * W)

            for j in (0, 1):
                asum, asq, amx, amn = accs[j]
                sl = pl.ds((c0 + j) * N, N)
                pltpu.sync_copy(asum, osum.at[sl])
                pltpu.sync_copy(asq, osq.at[sl])
                pltpu.sync_copy(amx, omx.at[sl])
                pltpu.sync_copy(amn, omn.at[sl])

    outs = k(aT.reshape(-1), bT.reshape(-1), cT.reshape(-1), src, dst)
    return [o.reshape(D, N) for o in outs]


def kernel(edge_index, r, atom_features, distances, graph_ids, af_table,
           W_atom, b_atom, W_dist, b_dist, ln_g, ln_b, W_edge, b_edge,
           W_pre0, b_pre0, W_post0, b_post0, W_pre1, b_pre1, W_post1, b_post1,
           W_out, b_out):
    src = edge_index[0].astype(jnp.int32)
    dst = edge_index[1].astype(jnp.int32)

    h, a0T, b0T = _node_prep(atom_features.astype(jnp.int32), distances,
                             af_table, W_atom, b_atom, W_dist, b_dist,
                             ln_g, ln_b, W_pre0[:D], W_pre0[D:2 * D])
    c0T, c1T = _edge_prep(r, W_edge, b_edge, W_pre0[2 * D:], b_pre0,
                          W_pre1[2 * D:], b_pre1)
    degp = _deg_sc(dst)

    s, q, x, n = _edge_stage(a0T, b0T, c0T, src, dst)
    h1 = _post(h, s, q, x, n, degp, W_post0, b_post0)
    a1T, b1T = _abt(h1, W_pre1[:D], W_pre1[D:2 * D])
    s, q, x, n = _edge_stage(a1T, b1T, c1T, src, dst)
    h2 = _post(h1, s, q, x, n, degp, W_post1, b_post1)
    return _readout(h2, graph_ids.astype(jnp.int32), W_out, b_out)


# speedup vs baseline: 3.3132x; 1.1104x over previous
import dataclasses
import functools

import jax
import jax.numpy as jnp
import numpy as np
from jax import lax
from jax.experimental import pallas as pl
from jax.experimental.pallas import tpu as pltpu
from jax.experimental.pallas import tpu_sc as plsc

N = 10000
E = 160000
D = 128
EF = 40
MAXN = 12
NG = 100
NATOM = 100
AENC = 200
DELTA = float(np.log(MAXN + 1.0))
NEG = np.float32(-3.0e38)

EB = 6400
W = 640
NWIN = E // W
VPW = W // 16
CHUNK = E // 32


def _node_prep_body(atom_ref, dist_ref, af_ref, Wa_ref, ba_ref, Wd_ref,
                    bd_ref, g_ref, b_ref, Wp1_ref, Wp2_ref,
                    h_ref, aT_ref, bT_ref):
    atom = atom_ref[...]
    onehot = (lax.broadcasted_iota(jnp.int32, (N, NATOM), 1)
              == atom[:, None]).astype(jnp.float32)
    table2 = jnp.dot(af_ref[...], Wa_ref[...],
                     preferred_element_type=jnp.float32)
    hp = jnp.dot(onehot, table2, preferred_element_type=jnp.float32)
    hp = hp + ba_ref[...][None, :]
    hp = hp + jnp.dot(dist_ref[...], Wd_ref[...],
                      preferred_element_type=jnp.float32) + bd_ref[...][None, :]
    mu = jnp.mean(hp, axis=1, keepdims=True)
    va = jnp.mean((hp - mu) * (hp - mu), axis=1, keepdims=True)
    h = (hp - mu) * lax.rsqrt(va + 1e-5) * g_ref[...][None, :] + b_ref[...][None, :]
    h_ref[...] = h
    aT_ref[...] = lax.dot_general(Wp1_ref[...], h, (((0,), (1,)), ((), ())),
                                  preferred_element_type=jnp.float32)
    bT_ref[...] = lax.dot_general(Wp2_ref[...], h, (((0,), (1,)), ((), ())),
                                  preferred_element_type=jnp.float32)


def _node_prep(atom, dist, af_table, W_atom, b_atom, W_dist, b_dist,
               ln_g, ln_b, Wp1, Wp2):
    return pl.pallas_call(
        _node_prep_body,
        out_shape=[jax.ShapeDtypeStruct((N, D), jnp.float32),
                   jax.ShapeDtypeStruct((D, N), jnp.float32),
                   jax.ShapeDtypeStruct((D, N), jnp.float32)],
    )(atom, dist, af_table, W_atom, b_atom, W_dist, b_dist, ln_g, ln_b,
      Wp1, Wp2)


_CENTERS = np.linspace(0.0, 1.0, EF).astype(np.float32)
_GAMMA = np.float32(1.0 / (_CENTERS[1] - _CENTERS[0]) ** 2)


def _edge_prep_body(r_ref, We_ref, be_ref, Wp30_ref, bp0_ref, Wp31_ref,
                    bp1_ref, c0_ref, c1_ref):
    r = r_ref[...]
    d2 = jnp.sum(r * r, axis=0)
    invd = 1.0 / jnp.sqrt(d2)
    centers = (lax.broadcasted_iota(jnp.int32, (EF, 1), 0)
               .astype(jnp.float32) * np.float32(1.0 / (EF - 1)))
    gT = jnp.exp(-_GAMMA * (invd[None, :] - centers) ** 2)
    for (Wp3_ref, bp_ref, c_ref) in ((Wp30_ref, bp0_ref, c0_ref),
                                     (Wp31_ref, bp1_ref, c1_ref)):
        U = jnp.dot(We_ref[...], Wp3_ref[...],
                    preferred_element_type=jnp.float32)
        cst = jnp.dot(be_ref[...][None, :], Wp3_ref[...],
                      preferred_element_type=jnp.float32)[0] + bp_ref[...]
        c_ref[...] = lax.dot_general(U, gT, (((0,), (0,)), ((), ())),
                                     preferred_element_type=jnp.float32) \
            + cst[:, None]


def _edge_prep(r, W_edge, b_edge, Wp30, bp0, Wp31, bp1):
    nblk = E // EB
    return pl.pallas_call(
        _edge_prep_body,
        grid=(nblk,),
        in_specs=[
            pl.BlockSpec((3, EB), lambda i: (0, i)),
            pl.BlockSpec((EF, D), lambda i: (0, 0)),
            pl.BlockSpec((D,), lambda i: (0,)),
            pl.BlockSpec((D, D), lambda i: (0, 0)),
            pl.BlockSpec((D,), lambda i: (0,)),
            pl.BlockSpec((D, D), lambda i: (0, 0)),
            pl.BlockSpec((D,), lambda i: (0,)),
        ],
        out_specs=[pl.BlockSpec((D, EB), lambda i: (0, i)),
                   pl.BlockSpec((D, EB), lambda i: (0, i))],
        out_shape=[jax.ShapeDtypeStruct((D, E), jnp.float32),
                   jax.ShapeDtypeStruct((D, E), jnp.float32)],
    )(r.T, W_edge, b_edge, Wp30, bp0, Wp31, bp1)


def _post_body(h_ref, s_ref, q_ref, x_ref, n_ref, degp_ref, Wq_ref, bq_ref,
               out_ref):
    deg = degp_ref[0, :] + degp_ref[1, :]
    degc = jnp.maximum(deg, 1.0)
    inv = 1.0 / degc
    meanT = s_ref[...] * inv[None, :]
    msqT = q_ref[...] * inv[None, :]
    varT = jnp.maximum(msqT - meanT * meanT, 0.0)
    pos = (deg > 0.0)[None, :]
    mxT = jnp.where(pos, x_ref[...], 0.0)
    mnT = jnp.where(pos, -n_ref[...], 0.0)
    att = (DELTA / jnp.log(degc + 1.0))[:, None]
    Wq = Wq_ref[...]

    def dgT(xT, k):
        return lax.dot_general(xT, Wq[D * k:D * (k + 1)],
                               (((0,), (0,)), ((), ())),
                               preferred_element_type=jnp.float32)

    out = jnp.dot(h_ref[...], Wq[:D], preferred_element_type=jnp.float32)
    out = out + dgT(meanT, 1) + att * dgT(meanT, 5)
    out = out + dgT(varT, 2) + att * dgT(varT, 6)
    out = out + dgT(mnT, 3) + att * dgT(mnT, 7)
    out = out + dgT(mxT, 4) + att * dgT(mxT, 8)
    out_ref[...] = out + bq_ref[...][None, :]


def _post(h, s, q, x, n, degp, Wq, bq):
    return pl.pallas_call(
        _post_body,
        out_shape=jax.ShapeDtypeStruct((N, D), jnp.float32),
    )(h, s, q, x, n, degp, Wq, bq)


def _abt_body(h_ref, Wp1_ref, Wp2_ref, aT_ref, bT_ref):
    h = h_ref[...]
    aT_ref[...] = lax.dot_general(Wp1_ref[...], h, (((0,), (1,)), ((), ())),
                                  preferred_element_type=jnp.float32)
    bT_ref[...] = lax.dot_general(Wp2_ref[...], h, (((0,), (1,)), ((), ())),
                                  preferred_element_type=jnp.float32)


def _abt(h, Wp1, Wp2):
    return pl.pallas_call(
        _abt_body,
        out_shape=[jax.ShapeDtypeStruct((D, N), jnp.float32),
                   jax.ShapeDtypeStruct((D, N), jnp.float32)],
    )(h, Wp1, Wp2)


def _readout_body(h_ref, gid_ref, w_ref, b_ref, o_ref):
    h = h_ref[...]
    gid = gid_ref[...]
    seg = lax.broadcasted_iota(jnp.int32, (NG, N), 0)
    onehot = (seg == gid[None, :]).astype(jnp.float32)
    s = jnp.dot(onehot, h, preferred_element_type=jnp.float32)
    cnt = jnp.maximum(jnp.sum(onehot, axis=1, keepdims=True), 1.0)
    pooled = s / cnt
    o_ref[...] = jnp.dot(pooled, w_ref[...],
                         preferred_element_type=jnp.float32) + b_ref[...][None, :]


def _readout(h, gid, W_out, b_out):
    return pl.pallas_call(
        _readout_body,
        out_shape=jax.ShapeDtypeStruct((NG, 1), jnp.float32),
    )(h, gid, W_out, b_out)


def _deg_sc(dst):
    mesh = plsc.VectorSubcoreMesh(core_axis_name="c", subcore_axis_name="s")

    @functools.partial(
        pl.kernel, mesh=mesh,
        out_type=jax.ShapeDtypeStruct((2 * N,), jnp.float32),
        scratch_types=[
            pltpu.VMEM((CHUNK,), jnp.int32),
            pltpu.VMEM((CHUNK,), jnp.float32),
            pltpu.VMEM_SHARED((N,), jnp.float32),
            pltpu.SemaphoreType.DMA,
        ])
    def k(dst_hbm, deg_hbm, dstv, onesv, shared_deg, sem):
        c = lax.axis_index("c")
        s = lax.axis_index("s")

        @pl.loop(0, CHUNK, step=16)
        def _fill(i):
            onesv[pl.ds(i, 16)] = jnp.zeros((16,), jnp.float32)

        @pl.when(s == 0)
        def _zero():
            pltpu.sync_copy(onesv, shared_deg.at[pl.ds(0, CHUNK)])
            pltpu.sync_copy(onesv, shared_deg.at[pl.ds(CHUNK, CHUNK)])

        @pl.loop(0, CHUNK, step=16)
        def _fill1(i):
            onesv[pl.ds(i, 16)] = jnp.ones((16,), jnp.float32)

        plsc.subcore_barrier()
        base = (c * 16 + s) * CHUNK
        pltpu.async_copy(dst_hbm.at[pl.ds(base, CHUNK)], dstv, sem).wait()
        pltpu.sync_copy(onesv, shared_deg.at[dstv], add=True)
        plsc.subcore_barrier()

        @pl.when(s == 0)
        def _out():
            for half in (0, 1):
                pltpu.sync_copy(shared_deg.at[pl.ds(half * CHUNK, CHUNK)],
                                onesv)
                pltpu.sync_copy(onesv,
                                deg_hbm.at[pl.ds(c * N + half * CHUNK, CHUNK)])

    return k(dst).reshape(2, N)


_IOTA16 = np.arange(16, dtype=np.int32)


def _edge_stage(aT, bT, cT, src, dst):
    mesh = plsc.VectorSubcoreMesh(core_axis_name="c", subcore_axis_name="s")

    vm = pltpu.VMEM
    scratch = ([vm((N,), jnp.float32)] * 8
               + [vm((N,), jnp.float32)] * 4
               + [vm((W,), jnp.int32)] * 4
               + [vm((W,), jnp.float32)] * 4
               + [vm((16,), jnp.int32), vm((16,), jnp.float32)]
               + [pltpu.SemaphoreType.DMA, pltpu.SemaphoreType.DMA])

    cp = pltpu.CompilerParams()
    if "needs_layout_passes" in pltpu.CompilerParams.__dataclass_fields__:
        cp = dataclasses.replace(cp, needs_layout_passes=False)

    @functools.partial(
        pl.kernel, mesh=mesh,
        out_type=[jax.ShapeDtypeStruct((D * N,), jnp.float32)] * 4,
        scratch_types=scratch, compiler_params=cp)
    def k(aT_hbm, bT_hbm, cT_hbm, src_hbm, dst_hbm,
          osum, osq, omx, omn,
          as0, as1, aq0, aq1, ax0, ax1, an0, an1,
          ac0, ac1, bc0, bc1,
          sb0, sb1, db0, db1,
          cb00, cb01, cb10, cb11,
          sk16, sv16, sem0, sem1):
        cidx = lax.axis_index("c")
        sidx = lax.axis_index("s")
        wid = sidx * 2 + cidx

        iota = lax.iota(jnp.int32, 16)
        sh1 = jnp.maximum(iota - 1, 0)
        sh2 = jnp.maximum(iota - 2, 0)
        sh4 = jnp.maximum(iota - 4, 0)
        sh8 = jnp.maximum(iota - 8, 0)
        shp1 = jnp.minimum(iota + 1, 15)
        ge1 = iota >= 1
        ge2 = iota >= 2
        ge4 = iota >= 4
        ge8 = iota >= 8
        is15 = iota == 15
        shifts = ((sh1, ge1), (sh2, ge2), (sh4, ge4), (sh8, ge8))

        accs = ((as0, aq0, ax0, an0), (as1, aq1, ax1, an1))
        acols = (ac0, ac1)
        bcols = (bc0, bc1)
        sbufs = (sb0, sb1)
        dbufs = (db0, db1)
        cbufs = ((cb00, cb01), (cb10, cb11))
        sems = (sem0, sem1)

        def issue(slot, w, c0):
            off = w * W
            sem = sems[slot]
            pltpu.async_copy(src_hbm.at[pl.ds(off, W)], sbufs[slot], sem)
            pltpu.async_copy(dst_hbm.at[pl.ds(off, W)], dbufs[slot], sem)
            pltpu.async_copy(cT_hbm.at[pl.ds(c0 * E + off, W)],
                             cbufs[slot][0], sem)
            pltpu.async_copy(cT_hbm.at[pl.ds((c0 + 1) * E + off, W)],
                             cbufs[slot][1], sem)

        def drain(slot):
            sem = sems[slot]
            pltpu.make_async_copy(src_hbm.at[pl.ds(0, W)], sbufs[slot],
                                  sem).wait()
            pltpu.make_async_copy(dst_hbm.at[pl.ds(0, W)], dbufs[slot],
                                  sem).wait()
            pltpu.make_async_copy(cT_hbm.at[pl.ds(0, W)], cbufs[slot][0],
                                  sem).wait()
            pltpu.make_async_copy(cT_hbm.at[pl.ds(0, W)], cbufs[slot][1],
                                  sem).wait()

        def process(slot, base_scalar):
            sb = sbufs[slot]
            db = dbufs[slot]

            @pl.loop(0, VPW, step=4)
            def _vec(v):
                work = []
                for u in range(4):
                    b2 = (v + u) * 16
                    dst16 = db[pl.ds(b2, 16)]
                    src16 = sb[pl.ds(b2, 16)]
                    for j in (0, 1):
                        a = plsc.load_gather(acols[j], [src16])
                        b = plsc.load_gather(bcols[j], [dst16])
                        cv = cbufs[slot][j][pl.ds(b2, 16)]
                        m = a + b + cv
                        sq = m * m
                        asum, asq, _, _ = accs[j]
                        plsc.addupdate_scatter(asum, [dst16], m)
                        plsc.addupdate_scatter(asq, [dst16], sq)
                        work.append((j, dst16, m, -m))
                news = []
                for j, dst16, m, mn in work:
                    _, _, amx, amn = accs[j]
                    newx = jnp.maximum(plsc.load_gather(amx, [dst16]), m)
                    plsc.store_scatter(amx, [dst16], newx)
                    newn = jnp.maximum(plsc.load_gather(amn, [dst16]), mn)
                    plsc.store_scatter(amn, [dst16], newn)
                    news.append((newx, newn))
                fails = []
                anyf = None
                for (j, dst16, m, mn), (newx, newn) in zip(work, news):
                    _, _, amx, amn = accs[j]
                    fx = plsc.load_gather(amx, [dst16]) < newx
                    fn = plsc.load_gather(amn, [dst16]) < newn
                    fails.append((fx, fn))
                    f = fx | fn
                    anyf = f if anyf is None else (anyf | f)

                @pl.when(jnp.any(anyf))
                def _retry():
                    def rcond(carry):
                        f = None
                        for fx, fn in carry:
                            g = fx | fn
                            f = g if f is None else (f | g)
                        return jnp.any(f)

                    def rbody(carry):
                        out = []
                        for (j, dst16, m, mn), (fx, fn) in zip(work, carry):
                            _, _, amx, amn = accs[j]
                            new = jnp.maximum(plsc.load_gather(amx, [dst16]),
                                              m)
                            plsc.store_scatter(amx, [dst16], new, mask=fx)
                            fx = fx & (plsc.load_gather(amx, [dst16]) < new)
                            new = jnp.maximum(plsc.load_gather(amn, [dst16]),
                                              mn)
                            plsc.store_scatter(amn, [dst16], new, mask=fn)
                            fn = fn & (plsc.load_gather(amn, [dst16]) < new)
                            out.append((fx, fn))
                        return tuple(out)

                    lax.while_loop(rcond, rbody, tuple(fails))

        for p in (0, 1):
            c0 = 2 * wid + 64 * p

            @pl.loop(0, N, step=16)
            def _init(i):
                z = jnp.zeros((16,), jnp.float32)
                ng = jnp.full((16,), NEG, jnp.float32)
                for j in (0, 1):
                    asum, asq, amx, amn = accs[j]
                    asum[pl.ds(i, 16)] = z
                    asq[pl.ds(i, 16)] = z
                    amx[pl.ds(i, 16)] = ng
                    amn[pl.ds(i, 16)] = ng

            for j in (0, 1):
                pltpu.async_copy(aT_hbm.at[pl.ds((c0 + j) * N, N)],
                                 acols[j], sems[0]).wait()
                pltpu.async_copy(bT_hbm.at[pl.ds((c0 + j) * N, N)],
                                 bcols[j], sems[0]).wait()

            issue(0, 0, c0)

            @pl.loop(0, NWIN, step=2)
            def _win(w):
                issue(1, w + 1, c0)
                drain(0)
                process(0, w * W)

                @pl.when(w + 2 < NWIN)
                def _nxt():
                    issue(0, w + 2, c0)

                drain(1)
                process(1, (w + 1) * W)

            for j in (0, 1):
                asum, asq, amx, amn = accs[j]
                sl = pl.ds((c0 + j) * N, N)
                pltpu.sync_copy(asum, osum.at[sl])
                pltpu.sync_copy(asq, osq.at[sl])
                pltpu.sync_copy(amx, omx.at[sl])
                pltpu.sync_copy(amn, omn.at[sl])

    outs = k(aT.reshape(-1), bT.reshape(-1), cT.reshape(-1), src, dst)
    return [o.reshape(D, N) for o in outs]


def kernel(edge_index, r, atom_features, distances, graph_ids, af_table,
           W_atom, b_atom, W_dist, b_dist, ln_g, ln_b, W_edge, b_edge,
           W_pre0, b_pre0, W_post0, b_post0, W_pre1, b_pre1, W_post1, b_post1,
           W_out, b_out):
    src = edge_index[0].astype(jnp.int32)
    dst = edge_index[1].astype(jnp.int32)

    h, a0T, b0T = _node_prep(atom_features.astype(jnp.int32), distances,
                             af_table, W_atom, b_atom, W_dist, b_dist,
                             ln_g, ln_b, W_pre0[:D], W_pre0[D:2 * D])
    c0T, c1T = _edge_prep(r, W_edge, b_edge, W_pre0[2 * D:], b_pre0,
                          W_pre1[2 * D:], b_pre1)
    degp = _deg_sc(dst)

    s, q, x, n = _edge_stage(a0T, b0T, c0T, src, dst)
    h1 = _post(h, s, q, x, n, degp, W_post0, b_post0)
    a1T, b1T = _abt(h1, W_pre1[:D], W_pre1[D:2 * D])
    s, q, x, n = _edge_stage(a1T, b1T, c1T, src, dst)
    h2 = _post(h1, s, q, x, n, degp, W_post1, b_post1)
    return _readout(h2, graph_ids.astype(jnp.int32), W_out, b_out)
